# baseline (speedup 1.0000x reference)
import jax
import jax.numpy as jnp
from jax import lax
from jax.experimental import pallas as pl
from jax.experimental.pallas import tpu as pltpu
from jax.experimental.pallas import tpu_sc as plsc

T, D, E, K, FF, FFS = 64, 1024, 64, 2, 512, 1024

_NSUB = 16
_TPS = T // _NSUB
_L = 16
_NCH = E // _L


def _logits_body(x_ref, gw_ref, out_ref):
    out_ref[...] = jnp.dot(x_ref[...], gw_ref[...].T,
                           preferred_element_type=jnp.float32)


def _shared_body(x_ref, sg_ref, su_ref, sd_ref, out_ref):
    xb = x_ref[...].astype(jnp.bfloat16)
    sg = jnp.dot(xb, sg_ref[...].astype(jnp.bfloat16),
                 preferred_element_type=jnp.float32)
    su = jnp.dot(xb, su_ref[...].astype(jnp.bfloat16),
                 preferred_element_type=jnp.float32)
    sh = jax.nn.silu(sg) * su
    out_ref[...] = jnp.dot(sh.astype(jnp.bfloat16),
                           sd_ref[...].astype(jnp.bfloat16),
                           preferred_element_type=jnp.float32)


def _sc_router(logits_hbm, bias_hbm, comb_hbm, act_hbm,
               scores_v, comb_v, bias_v, act_v):
    cid = lax.axis_index("c")
    sid = lax.axis_index("s")

    @pl.when(cid == 0)
    def _route():
        pltpu.sync_copy(logits_hbm.at[pl.ds(sid * _TPS, _TPS)], scores_v)
        pltpu.sync_copy(bias_hbm, bias_v)
        acc = [jnp.zeros((_L,), jnp.float32) for _ in range(_NCH)]
        for lt in range(_TPS):
            sig_c, sc_c, m_c, i_c = [], [], [], []
            for c in range(_NCH):
                v = scores_v[lt, pl.ds(c * _L, _L)]
                sig = 1.0 / (1.0 + jnp.exp(-v))
                scc = sig + bias_v[pl.ds(c * _L, _L)]
                gidx = lax.iota(jnp.int32, _L) + c * _L
                m = jnp.max(scc)
                i = jnp.min(jnp.where(scc == m, gidx, E))
                sig_c.append(sig), sc_c.append(scc)
                m_c.append(m), i_c.append(i)
            m1, i1 = m_c[0], i_c[0]
            for c in range(1, _NCH):
                better = (m_c[c] > m1) | ((m_c[c] == m1) & (i_c[c] < i1))
                m1 = jnp.where(better, m_c[c], m1)
                i1 = jnp.where(better, i_c[c], i1)
            m2 = jnp.float32(-3e38)
            i2 = jnp.int32(E)
            for c in range(_NCH):
                gidx = lax.iota(jnp.int32, _L) + c * _L
                scm = jnp.where(gidx == i1, -3e38, sc_c[c])
                m = jnp.max(scm)
                i = jnp.min(jnp.where(scm == m, gidx, E))
                better = (m > m2) | ((m == m2) & (i < i2))
                m2 = jnp.where(better, m, m2)
                i2 = jnp.where(better, i, i2)
            w1 = jnp.float32(0.0)
            w2 = jnp.float32(0.0)
            for c in range(_NCH):
                gidx = lax.iota(jnp.int32, _L) + c * _L
                w1 = w1 + jnp.sum(jnp.where(gidx == i1, sig_c[c], 0.0))
                w2 = w2 + jnp.sum(jnp.where(gidx == i2, sig_c[c], 0.0))
            den = w1 + w2
            for c in range(_NCH):
                gidx = lax.iota(jnp.int32, _L) + c * _L
                row = (jnp.where(gidx == i1, w1, 0.0)
                       + jnp.where(gidx == i2, w2, 0.0)) / den
                comb_v[lt, pl.ds(c * _L, _L)] = row
                acc[c] = jnp.maximum(acc[c], row)
        pltpu.sync_copy(comb_v, comb_hbm.at[pl.ds(sid * _TPS, _TPS)])
        for c in range(_NCH):
            act_v[pl.ds(c * _L, _L)] = acc[c]
        pltpu.sync_copy(act_v, act_hbm.at[sid])


def _sc_schedule(act_hbm, ids_hbm, big_v, ids_v):
    cid = lax.axis_index("c")
    sid = lax.axis_index("s")

    @pl.when((cid == 0) & (sid == 0))
    def _schedule():
        pltpu.sync_copy(act_hbm, big_v)
        fa = jnp.int32(E)
        carry = jnp.int32(-1)
        for c in range(_NCH):
            gidx = lax.iota(jnp.int32, _L) + c * _L
            u = jnp.zeros((_L,), jnp.float32)
            for r in range(_NSUB):
                u = jnp.maximum(u, big_v[r, pl.ds(c * _L, _L)])
            fa = jnp.minimum(fa, jnp.min(jnp.where(u > 0.0, gidx, E)))
            val = jnp.where(u > 0.0, gidx, -1)
            cm = jnp.maximum(plsc.cummax(val), carry)
            carry = jnp.max(cm)
            ids_v[pl.ds(c * _L, _L)] = cm
        for c in range(_NCH):
            w = ids_v[pl.ds(c * _L, _L)]
            ids_v[pl.ds(c * _L, _L)] = jnp.where(w < 0, fa, w)
        pltpu.sync_copy(ids_v, ids_hbm)


_NSH = 4
_SHC = FFS // _NSH


def _expert_step(ids_ref, comb_ref, x_ref, shared_ref,
                 wg_ref, wu_ref, wd_ref, out_ref):
    i = pl.program_id(0)
    e = ids_ref[i]

    @pl.when(i == 0)
    def _init():
        out_ref[...] = shared_ref[...]

    @pl.when(e == i)
    def _acc():
        xb = x_ref[...].astype(jnp.bfloat16)
        g = jnp.dot(xb, wg_ref[0].astype(jnp.bfloat16),
                    preferred_element_type=jnp.float32)
        u = jnp.dot(xb, wu_ref[0].astype(jnp.bfloat16),
                    preferred_element_type=jnp.float32)
        h = jax.nn.silu(g) * u
        lane = jax.lax.broadcasted_iota(jnp.int32, (T, E), 1)
        ce = jnp.sum(jnp.where(lane == i, comb_ref[...], 0.0), axis=1,
                     keepdims=True)
        out_ref[...] += jnp.dot((h * ce).astype(jnp.bfloat16),
                                wd_ref[0].astype(jnp.bfloat16),
                                preferred_element_type=jnp.float32)


def kernel(hidden_states, gate_w, w_gate, w_up, w_down, shared_gate,
           shared_up, shared_down, e_score_correction_bias):
    orig_shape = hidden_states.shape
    x = hidden_states.reshape(-1, orig_shape[-1])

    logits = pl.pallas_call(
        _logits_body,
        out_shape=jax.ShapeDtypeStruct((T, E), jnp.float32),
    )(x, gate_w)

    shared = pl.pallas_call(
        _shared_body,
        out_shape=jax.ShapeDtypeStruct((T, D), jnp.float32),
    )(x, shared_gate, shared_up, shared_down)

    mesh = plsc.VectorSubcoreMesh(core_axis_name="c", subcore_axis_name="s")
    comb, act = pl.kernel(
        _sc_router,
        out_type=(
            jax.ShapeDtypeStruct((T, E), jnp.float32),
            jax.ShapeDtypeStruct((_NSUB, E), jnp.float32),
        ),
        mesh=mesh,
        compiler_params=pltpu.CompilerParams(needs_layout_passes=False),
        scratch_types=[
            pltpu.VMEM((_TPS, E), jnp.float32),
            pltpu.VMEM((_TPS, E), jnp.float32),
            pltpu.VMEM((E,), jnp.float32),
            pltpu.VMEM((E,), jnp.float32),
        ],
    )(logits, e_score_correction_bias)
    ids = pl.kernel(
        _sc_schedule,
        out_type=jax.ShapeDtypeStruct((E,), jnp.int32),
        mesh=mesh,
        compiler_params=pltpu.CompilerParams(needs_layout_passes=False),
        scratch_types=[
            pltpu.VMEM((_NSUB, E), jnp.float32),
            pltpu.VMEM((E,), jnp.int32),
        ],
    )(act)

    grid_spec = pltpu.PrefetchScalarGridSpec(
        num_scalar_prefetch=1,
        grid=(E,),
        in_specs=[
            pl.BlockSpec((T, E), lambda i, ids: (0, 0)),
            pl.BlockSpec((T, D), lambda i, ids: (0, 0)),
            pl.BlockSpec((T, D), lambda i, ids: (0, 0)),
            pl.BlockSpec((1, D, FF), lambda i, ids: (ids[i], 0, 0)),
            pl.BlockSpec((1, D, FF), lambda i, ids: (ids[i], 0, 0)),
            pl.BlockSpec((1, FF, D), lambda i, ids: (ids[i], 0, 0)),
        ],
        out_specs=pl.BlockSpec((T, D), lambda i, ids: (0, 0)),
    )
    out = pl.pallas_call(
        _expert_step,
        grid_spec=grid_spec,
        out_shape=jax.ShapeDtypeStruct((T, D), jnp.float32),
    )(ids, comb, x, shared, w_gate, w_up, w_down)
    return out.reshape(orig_shape)


# speedup vs baseline: 1.0129x; 1.0129x over previous
import jax
import jax.numpy as jnp
from jax import lax
from jax.experimental import pallas as pl
from jax.experimental.pallas import tpu as pltpu
from jax.experimental.pallas import tpu_sc as plsc

T, D, E, K, FF, FFS = 64, 1024, 64, 2, 512, 1024

_NSUB = 16
_TPS = T // _NSUB
_L = 16
_NCH = E // _L


def _pre_body(x_ref, gw_ref, sg_ref, su_ref, sd_ref, logits_ref, shared_ref):
    logits_ref[...] = jnp.dot(x_ref[...], gw_ref[...].T,
                              preferred_element_type=jnp.float32)
    xb = x_ref[...].astype(jnp.bfloat16)
    sg = jnp.dot(xb, sg_ref[...].astype(jnp.bfloat16),
                 preferred_element_type=jnp.float32)
    su = jnp.dot(xb, su_ref[...].astype(jnp.bfloat16),
                 preferred_element_type=jnp.float32)
    sh = jax.nn.silu(sg) * su
    shared_ref[...] = jnp.dot(sh.astype(jnp.bfloat16),
                              sd_ref[...].astype(jnp.bfloat16),
                              preferred_element_type=jnp.float32)


def _ids_body(act_ref, ids_ref):
    actT = act_ref[...].T
    active_rows = jnp.sum(actT, axis=1, keepdims=True) > 0.0
    jj = jax.lax.broadcasted_iota(jnp.int32, (E, E), 0)
    ii = jax.lax.broadcasted_iota(jnp.int32, (E, E), 1)
    val = jnp.where((jj <= ii) & active_rows, jj, -1)
    eid_raw = jnp.max(val, axis=0, keepdims=True)
    jcol = jax.lax.broadcasted_iota(jnp.int32, (E, 1), 0)
    first_active = jnp.min(jnp.where(active_rows, jcol, E))
    ids_ref[...] = jnp.where(eid_raw < 0, first_active, eid_raw)


def _sc_router(logits_hbm, bias_hbm, comb_hbm, act_hbm,
               scores_v, comb_v, bias_v, act_v):
    cid = lax.axis_index("c")
    sid = lax.axis_index("s")

    @pl.when(cid == 0)
    def _route():
        pltpu.sync_copy(logits_hbm.at[pl.ds(sid * _TPS, _TPS)], scores_v)
        pltpu.sync_copy(bias_hbm, bias_v)
        acc = [jnp.zeros((_L,), jnp.float32) for _ in range(_NCH)]
        for lt in range(_TPS):
            sig_c, sc_c, m_c, i_c = [], [], [], []
            for c in range(_NCH):
                v = scores_v[lt, pl.ds(c * _L, _L)]
                sig = 1.0 / (1.0 + jnp.exp(-v))
                scc = sig + bias_v[pl.ds(c * _L, _L)]
                gidx = lax.iota(jnp.int32, _L) + c * _L
                m = jnp.max(scc)
                i = jnp.min(jnp.where(scc == m, gidx, E))
                sig_c.append(sig), sc_c.append(scc)
                m_c.append(m), i_c.append(i)
            m1, i1 = m_c[0], i_c[0]
            for c in range(1, _NCH):
                better = (m_c[c] > m1) | ((m_c[c] == m1) & (i_c[c] < i1))
                m1 = jnp.where(better, m_c[c], m1)
                i1 = jnp.where(better, i_c[c], i1)
            m2 = jnp.float32(-3e38)
            i2 = jnp.int32(E)
            for c in range(_NCH):
                gidx = lax.iota(jnp.int32, _L) + c * _L
                scm = jnp.where(gidx == i1, -3e38, sc_c[c])
                m = jnp.max(scm)
                i = jnp.min(jnp.where(scm == m, gidx, E))
                better = (m > m2) | ((m == m2) & (i < i2))
                m2 = jnp.where(better, m, m2)
                i2 = jnp.where(better, i, i2)
            w1 = jnp.float32(0.0)
            w2 = jnp.float32(0.0)
            for c in range(_NCH):
                gidx = lax.iota(jnp.int32, _L) + c * _L
                w1 = w1 + jnp.sum(jnp.where(gidx == i1, sig_c[c], 0.0))
                w2 = w2 + jnp.sum(jnp.where(gidx == i2, sig_c[c], 0.0))
            den = w1 + w2
            for c in range(_NCH):
                gidx = lax.iota(jnp.int32, _L) + c * _L
                row = (jnp.where(gidx == i1, w1, 0.0)
                       + jnp.where(gidx == i2, w2, 0.0)) / den
                comb_v[lt, pl.ds(c * _L, _L)] = row
                acc[c] = jnp.maximum(acc[c], row)
        pltpu.sync_copy(comb_v, comb_hbm.at[pl.ds(sid * _TPS, _TPS)])
        for c in range(_NCH):
            act_v[pl.ds(c * _L, _L)] = acc[c]
        pltpu.sync_copy(act_v, act_hbm.at[sid])


_NSH = 4
_SHC = FFS // _NSH


def _expert_step(ids_ref, comb_ref, x_ref, shared_ref,
                 wg_ref, wu_ref, wd_ref, out_ref):
    i = pl.program_id(0)
    e = ids_ref[i]

    @pl.when(i == 0)
    def _init():
        out_ref[...] = shared_ref[...]

    @pl.when(e == i)
    def _acc():
        xb = x_ref[...].astype(jnp.bfloat16)
        g = jnp.dot(xb, wg_ref[0].astype(jnp.bfloat16),
                    preferred_element_type=jnp.float32)
        u = jnp.dot(xb, wu_ref[0].astype(jnp.bfloat16),
                    preferred_element_type=jnp.float32)
        h = jax.nn.silu(g) * u
        lane = jax.lax.broadcasted_iota(jnp.int32, (T, E), 1)
        ce = jnp.sum(jnp.where(lane == i, comb_ref[...], 0.0), axis=1,
                     keepdims=True)
        out_ref[...] += jnp.dot((h * ce).astype(jnp.bfloat16),
                                wd_ref[0].astype(jnp.bfloat16),
                                preferred_element_type=jnp.float32)


def kernel(hidden_states, gate_w, w_gate, w_up, w_down, shared_gate,
           shared_up, shared_down, e_score_correction_bias):
    orig_shape = hidden_states.shape
    x = hidden_states.reshape(-1, orig_shape[-1])

    logits, shared = pl.pallas_call(
        _pre_body,
        out_shape=[
            jax.ShapeDtypeStruct((T, E), jnp.float32),
            jax.ShapeDtypeStruct((T, D), jnp.float32),
        ],
    )(x, gate_w, shared_gate, shared_up, shared_down)

    mesh = plsc.VectorSubcoreMesh(core_axis_name="c", subcore_axis_name="s")
    comb, act = pl.kernel(
        _sc_router,
        out_type=(
            jax.ShapeDtypeStruct((T, E), jnp.float32),
            jax.ShapeDtypeStruct((_NSUB, E), jnp.float32),
        ),
        mesh=mesh,
        compiler_params=pltpu.CompilerParams(needs_layout_passes=False),
        scratch_types=[
            pltpu.VMEM((_TPS, E), jnp.float32),
            pltpu.VMEM((_TPS, E), jnp.float32),
            pltpu.VMEM((E,), jnp.float32),
            pltpu.VMEM((E,), jnp.float32),
        ],
    )(logits, e_score_correction_bias)
    ids = pl.pallas_call(
        _ids_body,
        out_shape=jax.ShapeDtypeStruct((1, E), jnp.int32),
    )(act)

    grid_spec = pltpu.PrefetchScalarGridSpec(
        num_scalar_prefetch=1,
        grid=(E,),
        in_specs=[
            pl.BlockSpec((T, E), lambda i, ids: (0, 0)),
            pl.BlockSpec((T, D), lambda i, ids: (0, 0)),
            pl.BlockSpec((T, D), lambda i, ids: (0, 0)),
            pl.BlockSpec((1, D, FF), lambda i, ids: (ids[i], 0, 0)),
            pl.BlockSpec((1, D, FF), lambda i, ids: (ids[i], 0, 0)),
            pl.BlockSpec((1, FF, D), lambda i, ids: (ids[i], 0, 0)),
        ],
        out_specs=pl.BlockSpec((T, D), lambda i, ids: (0, 0)),
    )
    out = pl.pallas_call(
        _expert_step,
        grid_spec=grid_spec,
        out_shape=jax.ShapeDtypeStruct((T, D), jnp.float32),
    )(ids.reshape(E), comb, x, shared, w_gate, w_up, w_down)
    return out.reshape(orig_shape)
